# baseline (speedup 1.0000x reference)
import functools

import jax
import jax.numpy as jnp
from jax import lax
from jax.experimental import pallas as pl
from jax.experimental.pallas import tpu as pltpu
from jax.experimental.pallas import tpu_sc as plsc

EPS = 1e-12
SUB = 128
SUBS_PER_CHUNK = 4
CHUNK = SUB * SUBS_PER_CHUNK
HIGHEST = jax.lax.Precision.HIGHEST


def _transpose_pack_body(x_ref, e_ref, o_ref):
    D = x_ref.shape[0]
    xt = lax.dot_general(x_ref[...], e_ref[...],
                         (((0,), (0,)), ((), ())), precision=HIGHEST)
    y3 = xt.reshape(xt.shape[0] // 2, 2, D)
    o_ref[:, :D] = y3[:, 0, :]
    o_ref[:, D:] = y3[:, 1, :]


def _tc_transpose_pack(tT, eye):
    D, V = tT.shape
    BN = 2048
    return pl.pallas_call(
        _transpose_pack_body,
        grid=(pl.cdiv(V, BN),),
        in_specs=[
            pl.BlockSpec((D, BN), lambda j: (0, j)),
            pl.BlockSpec((D, D), lambda j: (0, 0)),
        ],
        out_specs=pl.BlockSpec((BN // 2, 2 * D), lambda j: (j, 0)),
        out_shape=jax.ShapeDtypeStruct((V // 2, 2 * D), jnp.float32),
    )(tT, eye)


def _sc_gather(table2, idx2d, n_rows):
    n_idx_rows, _ = idx2d.shape
    D = table2.shape[1]
    info = plsc.get_sparse_core_info()
    NC, NS = info.num_cores, info.num_subcores
    NW = NC * NS
    rows_pw = n_idx_rows // NW
    cpw = rows_pw // SUBS_PER_CHUNK
    assert cpw % 2 == 0
    mesh = plsc.VectorSubcoreMesh(core_axis_name="c", subcore_axis_name="s")

    @functools.partial(
        pl.kernel,
        mesh=mesh,
        out_type=jax.ShapeDtypeStruct((n_rows, D), jnp.float32),
        scratch_types=[
            pltpu.VMEM((SUBS_PER_CHUNK, SUB), jnp.int32),
            pltpu.VMEM((SUBS_PER_CHUNK, SUB), jnp.int32),
            pltpu.VMEM((CHUNK, D), jnp.float32),
            pltpu.VMEM((CHUNK, D), jnp.float32),
            pltpu.SemaphoreType.DMA,
            pltpu.SemaphoreType.DMA,
        ],
        compiler_params=pltpu.CompilerParams(use_tc_tiling_on_sc=False),
    )
    def k(tab_hbm, idx_hbm, out_hbm, idx0, idx1, rows0, rows1, sem0, sem1):
        wid = lax.axis_index("s") * NC + lax.axis_index("c")
        base = wid * rows_pw

        def fire(c, idx_v, rows_v, sem):
            r0 = base + c * SUBS_PER_CHUNK
            pltpu.sync_copy(idx_hbm.at[pl.ds(r0, SUBS_PER_CHUNK)], idx_v)
            for j in range(SUBS_PER_CHUNK):
                pltpu.async_copy(
                    tab_hbm.at[idx_v.at[j]],
                    rows_v.at[pl.ds(j * SUB, SUB)],
                    sem,
                )

        def drain_write(c, idx_v, rows_v, sem):
            for j in range(SUBS_PER_CHUNK):
                pltpu.make_async_copy(
                    tab_hbm.at[idx_v.at[j]],
                    rows_v.at[pl.ds(j * SUB, SUB)],
                    sem,
                ).wait()
            flat = (base + c * SUBS_PER_CHUNK) * SUB
            pltpu.sync_copy(rows_v, out_hbm.at[pl.ds(flat, CHUNK)])

        def body(i, carry):
            c_even = 2 * i
            fire(c_even, idx0, rows0, sem0)

            @pl.when(i > 0)
            def _():
                drain_write(c_even - 1, idx1, rows1, sem1)

            fire(c_even + 1, idx1, rows1, sem1)
            drain_write(c_even, idx0, rows0, sem0)
            return carry

        lax.fori_loop(0, cpw // 2, body, 0)
        drain_write(cpw - 1, idx1, rows1, sem1)

    return k(table2, idx2d)


def _ln_elu_t_body(x_ref, w_ref, b_ref, e_ref, o_ref):
    c = pl.program_id(1)
    hi = (c % 2) == 1
    x = x_ref[...]
    D = o_ref.shape[1]
    xh = jnp.where(hi, x[:, D:], x[:, :D])
    inv = 1.0 / D
    u = jnp.sum(xh, axis=-1, keepdims=True) * inv
    xc = xh - u
    v = jnp.sum(xc * xc, axis=-1, keepdims=True) * inv
    y = xc * lax.rsqrt(v + EPS)
    y = y * w_ref[...] + b_ref[...]
    y = jnp.where(y > 0, y, jnp.exp(jnp.minimum(y, 0.0)) - 1.0)
    m = lax.dot_general(y, e_ref[...],
                        (((0,), (0,)), ((), ())), precision=HIGHEST)
    o_ref[...] = m.reshape(o_ref.shape)


def _tc_ln_elu_t(g2, w, b, eye, S, B, D):
    rows_per_s = B // 2
    blk = 128
    nblk = rows_per_s // blk
    return pl.pallas_call(
        _ln_elu_t_body,
        grid=(S, 2 * nblk),
        in_specs=[
            pl.BlockSpec((blk, 2 * D), lambda s, c: (s * nblk + c // 2, 0)),
            pl.BlockSpec((1, D), lambda s, c: (0, 0)),
            pl.BlockSpec((1, D), lambda s, c: (0, 0)),
            pl.BlockSpec((blk, blk), lambda s, c: (0, 0)),
        ],
        out_specs=pl.BlockSpec(
            (1, D, blk), lambda s, c: (s, 0, (c % 2) * nblk + c // 2)
        ),
        out_shape=jax.ShapeDtypeStruct((S, D, B), jnp.float32),
    )(g2, w.reshape(1, D), b.reshape(1, D), eye)


def kernel(sequence, table, ln_weight, ln_bias):
    B, S = sequence.shape
    V, D = table.shape
    n_rows = B * S

    eye64 = jnp.eye(D, dtype=jnp.float32)
    eye128 = jnp.eye(128, dtype=jnp.float32)

    tpack = _tc_transpose_pack(table.T, eye64)
    table2 = tpack.reshape(V, D)

    seqT = sequence.T
    idx_perm = seqT.reshape(S, 2, B // 2).transpose(0, 2, 1)
    idx2d = idx_perm.reshape(-1, SUB).astype(jnp.int32)

    g = _sc_gather(table2, idx2d, n_rows)
    g2 = g.reshape(n_rows // 2, 2 * D)

    out_t = _tc_ln_elu_t(g2, ln_weight, ln_bias, eye128, S, B, D)
    return out_t.transpose(2, 0, 1)


# speedup vs baseline: 1.9270x; 1.9270x over previous
import functools

import jax
import jax.numpy as jnp
from jax import lax
from jax.experimental import pallas as pl
from jax.experimental.pallas import tpu as pltpu
from jax.experimental.pallas import tpu_sc as plsc

EPS = 1e-12
SUB = 128
SUBS_PER_CHUNK = 4
CHUNK = SUB * SUBS_PER_CHUNK
HIGHEST = jax.lax.Precision.HIGHEST


def _transpose_pack_body(x_ref, o_ref):
    D = x_ref.shape[0]
    y = x_ref[...].T
    y3 = y.reshape(y.shape[0] // 2, 2, D)
    o_ref[:, :D] = y3[:, 0, :]
    o_ref[:, D:] = y3[:, 1, :]


def _tc_transpose_pack(tT):
    D, V = tT.shape
    BN = 2048
    return pl.pallas_call(
        _transpose_pack_body,
        grid=(pl.cdiv(V, BN),),
        in_specs=[pl.BlockSpec((D, BN), lambda j: (0, j))],
        out_specs=pl.BlockSpec((BN // 2, 2 * D), lambda j: (j, 0)),
        out_shape=jax.ShapeDtypeStruct((V // 2, 2 * D), jnp.float32),
    )(tT)


def _sc_gather(table2, idx2d, n_rows):
    n_idx_rows, _ = idx2d.shape
    D = table2.shape[1]
    info = plsc.get_sparse_core_info()
    NC, NS = info.num_cores, info.num_subcores
    NW = NC * NS
    rows_pw = n_idx_rows // NW
    cpw = rows_pw // SUBS_PER_CHUNK
    assert cpw % 2 == 0
    mesh = plsc.VectorSubcoreMesh(core_axis_name="c", subcore_axis_name="s")

    @functools.partial(
        pl.kernel,
        mesh=mesh,
        out_type=jax.ShapeDtypeStruct((n_rows, D), jnp.float32),
        scratch_types=[
            pltpu.VMEM((SUBS_PER_CHUNK, SUB), jnp.int32),
            pltpu.VMEM((SUBS_PER_CHUNK, SUB), jnp.int32),
            pltpu.VMEM((CHUNK, D), jnp.float32),
            pltpu.VMEM((CHUNK, D), jnp.float32),
            pltpu.SemaphoreType.DMA,
            pltpu.SemaphoreType.DMA,
        ],
        compiler_params=pltpu.CompilerParams(use_tc_tiling_on_sc=False),
    )
    def k(tab_hbm, idx_hbm, out_hbm, idx0, idx1, rows0, rows1, sem0, sem1):
        wid = lax.axis_index("s") * NC + lax.axis_index("c")
        base = wid * rows_pw

        def fire(c, idx_v, rows_v, sem):
            r0 = base + c * SUBS_PER_CHUNK
            pltpu.sync_copy(idx_hbm.at[pl.ds(r0, SUBS_PER_CHUNK)], idx_v)
            for j in range(SUBS_PER_CHUNK):
                pltpu.async_copy(
                    tab_hbm.at[idx_v.at[j]],
                    rows_v.at[pl.ds(j * SUB, SUB)],
                    sem,
                )

        def drain_write(c, idx_v, rows_v, sem):
            for j in range(SUBS_PER_CHUNK):
                pltpu.make_async_copy(
                    tab_hbm.at[idx_v.at[j]],
                    rows_v.at[pl.ds(j * SUB, SUB)],
                    sem,
                ).wait()
            flat = (base + c * SUBS_PER_CHUNK) * SUB
            pltpu.sync_copy(rows_v, out_hbm.at[pl.ds(flat, CHUNK)])

        def body(i, carry):
            c_even = 2 * i
            fire(c_even, idx0, rows0, sem0)

            @pl.when(i > 0)
            def _():
                drain_write(c_even - 1, idx1, rows1, sem1)

            fire(c_even + 1, idx1, rows1, sem1)
            drain_write(c_even, idx0, rows0, sem0)
            return carry

        lax.fori_loop(0, cpw // 2, body, 0)
        drain_write(cpw - 1, idx1, rows1, sem1)

    return k(table2, idx2d)


def _ln_elu_t_body(x_ref, w_ref, b_ref, p_ref, o_ref):
    x = x_ref[...]
    D = o_ref.shape[1]
    inv = 1.0 / D
    w = w_ref[...]
    b = b_ref[...]

    def norm(xh):
        u = jnp.sum(xh, axis=-1, keepdims=True) * inv
        xc = xh - u
        v = jnp.sum(xc * xc, axis=-1, keepdims=True) * inv
        y = xc * lax.rsqrt(v + EPS)
        y = y * w + b
        return jnp.where(y > 0, y, jnp.exp(jnp.minimum(y, 0.0)) - 1.0)

    ycat = jnp.concatenate([norm(x[:, :D]), norm(x[:, D:])], axis=0)
    m = lax.dot_general(ycat, p_ref[...],
                        (((0,), (0,)), ((), ())), precision=HIGHEST)
    o_ref[...] = m.reshape(o_ref.shape)


def _tc_ln_elu_t(g2, w, b, perm, S, B, D):
    blk = 128
    nblk = (B // 2) // blk
    return pl.pallas_call(
        _ln_elu_t_body,
        grid=(S, nblk),
        in_specs=[
            pl.BlockSpec((blk, 2 * D), lambda s, c: (s * nblk + c, 0)),
            pl.BlockSpec((1, D), lambda s, c: (0, 0)),
            pl.BlockSpec((1, D), lambda s, c: (0, 0)),
            pl.BlockSpec((2 * blk, 2 * blk), lambda s, c: (0, 0)),
        ],
        out_specs=pl.BlockSpec((1, D, 2 * blk), lambda s, c: (s, 0, c)),
        out_shape=jax.ShapeDtypeStruct((S, D, B), jnp.float32),
    )(g2, w.reshape(1, D), b.reshape(1, D), perm)


def kernel(sequence, table, ln_weight, ln_bias):
    B, S = sequence.shape
    V, D = table.shape
    n_rows = B * S

    tpack = _tc_transpose_pack(table.T)
    table2 = tpack.reshape(V, D)

    idx2d = sequence.T.reshape(-1, SUB).astype(jnp.int32)

    g = _sc_gather(table2, idx2d, n_rows)
    g2 = g.reshape(n_rows // 2, 2 * D)

    q = jnp.arange(256, dtype=jnp.int32)
    k = jnp.arange(256, dtype=jnp.int32)[:, None]
    perm = ((q[None, :] == jnp.where(k < 128, 2 * k, 2 * (k - 128) + 1))
            .astype(jnp.float32))

    out_t = _tc_ln_elu_t(g2, ln_weight, ln_bias, perm, S, B, D)
    return out_t.transpose(2, 0, 1)


# speedup vs baseline: 4.6668x; 2.4218x over previous
import functools

import jax
import jax.numpy as jnp
from jax import lax
from jax.experimental import pallas as pl
from jax.experimental.pallas import tpu as pltpu
from jax.experimental.pallas import tpu_sc as plsc

EPS = 1e-12
SUB = 128
SUBS_PER_CHUNK = 4
CHUNK = SUB * SUBS_PER_CHUNK


def _transpose_pack_body(x_ref, o_ref):
    D = x_ref.shape[0]
    y = x_ref[...].T
    y3 = y.reshape(y.shape[0] // 2, 2, D)
    o_ref[:, :D] = y3[:, 0, :]
    o_ref[:, D:] = y3[:, 1, :]


def _tc_transpose_pack(tT):
    D, V = tT.shape
    BN = 2048
    return pl.pallas_call(
        _transpose_pack_body,
        grid=(pl.cdiv(V, BN),),
        in_specs=[pl.BlockSpec((D, BN), lambda j: (0, j))],
        out_specs=pl.BlockSpec((BN // 2, 2 * D), lambda j: (j, 0)),
        out_shape=jax.ShapeDtypeStruct((V // 2, 2 * D), jnp.float32),
    )(tT)


def _sc_gather(table2, idx2d, n_rows):
    n_idx_rows, _ = idx2d.shape
    D = table2.shape[1]
    info = plsc.get_sparse_core_info()
    NC, NS = info.num_cores, info.num_subcores
    NW = NC * NS
    rows_pw = n_idx_rows // NW
    cpw = rows_pw // SUBS_PER_CHUNK
    assert cpw % 2 == 0
    mesh = plsc.VectorSubcoreMesh(core_axis_name="c", subcore_axis_name="s")

    @functools.partial(
        pl.kernel,
        mesh=mesh,
        out_type=jax.ShapeDtypeStruct((n_rows, D), jnp.float32),
        scratch_types=[
            pltpu.VMEM((SUBS_PER_CHUNK, SUB), jnp.int32),
            pltpu.VMEM((SUBS_PER_CHUNK, SUB), jnp.int32),
            pltpu.VMEM((CHUNK, D), jnp.float32),
            pltpu.VMEM((CHUNK, D), jnp.float32),
            pltpu.SemaphoreType.DMA,
            pltpu.SemaphoreType.DMA,
        ],
        compiler_params=pltpu.CompilerParams(use_tc_tiling_on_sc=False),
    )
    def k(tab_hbm, idx_hbm, out_hbm, idx0, idx1, rows0, rows1, sem0, sem1):
        wid = lax.axis_index("s") * NC + lax.axis_index("c")
        base = wid * rows_pw

        def fire(c, idx_v, rows_v, sem):
            r0 = base + c * SUBS_PER_CHUNK
            pltpu.sync_copy(idx_hbm.at[pl.ds(r0, SUBS_PER_CHUNK)], idx_v)
            for j in range(SUBS_PER_CHUNK):
                pltpu.async_copy(
                    tab_hbm.at[idx_v.at[j]],
                    rows_v.at[pl.ds(j * SUB, SUB)],
                    sem,
                )

        def drain_write(c, idx_v, rows_v, sem):
            for j in range(SUBS_PER_CHUNK):
                pltpu.make_async_copy(
                    tab_hbm.at[idx_v.at[j]],
                    rows_v.at[pl.ds(j * SUB, SUB)],
                    sem,
                ).wait()
            flat = (base + c * SUBS_PER_CHUNK) * SUB
            pltpu.sync_copy(rows_v, out_hbm.at[pl.ds(flat, CHUNK)])

        def body(i, carry):
            c_even = 2 * i
            fire(c_even, idx0, rows0, sem0)

            @pl.when(i > 0)
            def _():
                drain_write(c_even - 1, idx1, rows1, sem1)

            fire(c_even + 1, idx1, rows1, sem1)
            drain_write(c_even, idx0, rows0, sem0)
            return carry

        lax.fori_loop(0, cpw // 2, body, 0)
        drain_write(cpw - 1, idx1, rows1, sem1)

    return k(table2, idx2d)


def _ln_elu_body(x_ref, w_ref, b_ref, sel_ref, bc_ref, o_ref):
    x = x_ref[...]
    sel = sel_ref[...]
    bc = bc_ref[...]
    D = o_ref.shape[-1]
    inv = 1.0 / D
    sums = jax.lax.dot(x, sel, preferred_element_type=jnp.float32)
    u = jax.lax.dot(sums * inv, bc, preferred_element_type=jnp.float32)
    xc = x - u
    sq = jax.lax.dot(xc * xc, sel, preferred_element_type=jnp.float32)
    v = jax.lax.dot(sq * inv, bc, preferred_element_type=jnp.float32)
    y = xc * lax.rsqrt(v + EPS)
    y = y * w_ref[...] + b_ref[...]
    y = jnp.where(y > 0, y, jnp.exp(jnp.minimum(y, 0.0)) - 1.0)
    o_ref[::2, :] = y[:, :D]
    o_ref[1::2, :] = y[:, D:]


def _tc_ln_elu(x2, w2, b2, sel, bc):
    N2, L = x2.shape
    R = 1024
    return pl.pallas_call(
        _ln_elu_body,
        grid=(N2 // R,),
        in_specs=[
            pl.BlockSpec((R, L), lambda i: (i, 0)),
            pl.BlockSpec((1, L), lambda i: (0, 0)),
            pl.BlockSpec((1, L), lambda i: (0, 0)),
            pl.BlockSpec((L, 2), lambda i: (0, 0)),
            pl.BlockSpec((2, L), lambda i: (0, 0)),
        ],
        out_specs=pl.BlockSpec((2 * R, L // 2), lambda i: (i, 0)),
        out_shape=jax.ShapeDtypeStruct((2 * N2, L // 2), jnp.float32),
    )(x2, w2, b2, sel, bc)


def kernel(sequence, table, ln_weight, ln_bias):
    B, S = sequence.shape
    V, D = table.shape
    n_rows = B * S

    tpack = _tc_transpose_pack(table.T)
    table2 = tpack.reshape(V, D)

    idx2d = sequence.astype(jnp.int32).reshape(-1, SUB)
    g = _sc_gather(table2, idx2d, n_rows)
    g2 = g.reshape(n_rows // 2, 2 * D)

    half = jnp.arange(2 * D, dtype=jnp.int32) >= D
    sel = jnp.stack([1.0 - half.astype(jnp.float32),
                     half.astype(jnp.float32)], axis=1)
    bc = sel.T
    w2 = jnp.concatenate([ln_weight, ln_weight]).reshape(1, 2 * D)
    b2 = jnp.concatenate([ln_bias, ln_bias]).reshape(1, 2 * D)
    out = _tc_ln_elu(g2, w2, b2, sel, bc)
    return out.reshape(B, S, D)


# speedup vs baseline: 5.7287x; 1.2275x over previous
import functools

import jax
import jax.numpy as jnp
from jax import lax
from jax.experimental import pallas as pl
from jax.experimental.pallas import tpu as pltpu
from jax.experimental.pallas import tpu_sc as plsc

EPS = 1e-12
SUB = 128
SUBS_PER_CHUNK = 4
CHUNK = SUB * SUBS_PER_CHUNK


def _transpose_pack_body(x_ref, o_ref):
    D = x_ref.shape[0]
    y = x_ref[...].T
    y3 = y.reshape(y.shape[0] // 2, 2, D)
    o_ref[:, :D] = y3[:, 0, :]
    o_ref[:, D:] = y3[:, 1, :]


def _tc_transpose_pack(tT):
    D, V = tT.shape
    BN = 4096
    return pl.pallas_call(
        _transpose_pack_body,
        grid=(pl.cdiv(V, BN),),
        in_specs=[pl.BlockSpec((D, BN), lambda j: (0, j))],
        out_specs=pl.BlockSpec((BN // 2, 2 * D), lambda j: (j, 0)),
        out_shape=jax.ShapeDtypeStruct((V // 2, 2 * D), jnp.float32),
    )(tT)


def _sc_gather(table2, idx2d, n_rows):
    n_idx_rows, _ = idx2d.shape
    D = table2.shape[1]
    info = plsc.get_sparse_core_info()
    NC, NS = info.num_cores, info.num_subcores
    NW = NC * NS
    rows_pw = n_idx_rows // NW
    cpw = rows_pw // SUBS_PER_CHUNK
    assert cpw % 2 == 0
    mesh = plsc.VectorSubcoreMesh(core_axis_name="c", subcore_axis_name="s")

    @functools.partial(
        pl.kernel,
        mesh=mesh,
        out_type=jax.ShapeDtypeStruct((n_rows, D), jnp.float32),
        scratch_types=[
            pltpu.VMEM((SUBS_PER_CHUNK, SUB), jnp.int32),
            pltpu.VMEM((SUBS_PER_CHUNK, SUB), jnp.int32),
            pltpu.VMEM((CHUNK, D), jnp.float32),
            pltpu.VMEM((CHUNK, D), jnp.float32),
            pltpu.SemaphoreType.DMA,
            pltpu.SemaphoreType.DMA,
        ],
        compiler_params=pltpu.CompilerParams(use_tc_tiling_on_sc=False),
    )
    def k(tab_hbm, idx_hbm, out_hbm, idx0, idx1, rows0, rows1, sem0, sem1):
        wid = lax.axis_index("s") * NC + lax.axis_index("c")
        base = wid * rows_pw

        def fire(c, idx_v, rows_v, sem):
            r0 = base + c * SUBS_PER_CHUNK
            pltpu.sync_copy(idx_hbm.at[pl.ds(r0, SUBS_PER_CHUNK)], idx_v)
            for j in range(SUBS_PER_CHUNK):
                pltpu.async_copy(
                    tab_hbm.at[idx_v.at[j]],
                    rows_v.at[pl.ds(j * SUB, SUB)],
                    sem,
                )

        def drain_write(c, idx_v, rows_v, sem):
            for j in range(SUBS_PER_CHUNK):
                pltpu.make_async_copy(
                    tab_hbm.at[idx_v.at[j]],
                    rows_v.at[pl.ds(j * SUB, SUB)],
                    sem,
                ).wait()
            flat = (base + c * SUBS_PER_CHUNK) * SUB
            pltpu.sync_copy(rows_v, out_hbm.at[pl.ds(flat, CHUNK)])

        def body(i, carry):
            c_even = 2 * i
            fire(c_even, idx0, rows0, sem0)

            @pl.when(i > 0)
            def _():
                drain_write(c_even - 1, idx1, rows1, sem1)

            fire(c_even + 1, idx1, rows1, sem1)
            drain_write(c_even, idx0, rows0, sem0)
            return carry

        lax.fori_loop(0, cpw // 2, body, 0)
        drain_write(cpw - 1, idx1, rows1, sem1)

    return k(table2, idx2d)


def _ln_elu_body(x_ref, w_ref, b_ref, sel_ref, bc_ref, o_ref):
    x = x_ref[...]
    sel = sel_ref[...]
    bc = bc_ref[...]
    D = o_ref.shape[-1]
    inv = 1.0 / D
    sums = jax.lax.dot(x, sel, preferred_element_type=jnp.float32)
    u = jax.lax.dot(sums * inv, bc, preferred_element_type=jnp.float32)
    xc = x - u
    sq = jax.lax.dot(xc * xc, sel, preferred_element_type=jnp.float32)
    v = jax.lax.dot(sq * inv, bc, preferred_element_type=jnp.float32)
    y = xc * lax.rsqrt(v + EPS)
    y = y * w_ref[...] + b_ref[...]
    y = jnp.where(y > 0, y, jnp.exp(jnp.minimum(y, 0.0)) - 1.0)
    o_ref[::2, :] = y[:, :D]
    o_ref[1::2, :] = y[:, D:]


def _tc_ln_elu(x2, w2, b2, sel, bc):
    N2, L = x2.shape
    R = 2048
    return pl.pallas_call(
        _ln_elu_body,
        grid=(N2 // R,),
        in_specs=[
            pl.BlockSpec((R, L), lambda i: (i, 0)),
            pl.BlockSpec((1, L), lambda i: (0, 0)),
            pl.BlockSpec((1, L), lambda i: (0, 0)),
            pl.BlockSpec((L, 2), lambda i: (0, 0)),
            pl.BlockSpec((2, L), lambda i: (0, 0)),
        ],
        out_specs=pl.BlockSpec((2 * R, L // 2), lambda i: (i, 0)),
        out_shape=jax.ShapeDtypeStruct((2 * N2, L // 2), jnp.float32),
    )(x2, w2, b2, sel, bc)


def kernel(sequence, table, ln_weight, ln_bias):
    B, S = sequence.shape
    V, D = table.shape
    n_rows = B * S

    tpack = _tc_transpose_pack(table.T)
    table2 = tpack.reshape(V, D)

    idx2d = sequence.astype(jnp.int32).reshape(-1, SUB)
    g = _sc_gather(table2, idx2d, n_rows)
    g2 = g.reshape(n_rows // 2, 2 * D)

    half = jnp.arange(2 * D, dtype=jnp.int32) >= D
    sel = jnp.stack([1.0 - half.astype(jnp.float32),
                     half.astype(jnp.float32)], axis=1)
    bc = sel.T
    w2 = jnp.concatenate([ln_weight, ln_weight]).reshape(1, 2 * D)
    b2 = jnp.concatenate([ln_bias, ln_bias]).reshape(1, 2 * D)
    out = _tc_ln_elu(g2, w2, b2, sel, bc)
    return out.reshape(B, S, D)


# speedup vs baseline: 6.3435x; 1.1073x over previous
import functools

import jax
import jax.numpy as jnp
from jax import lax
from jax.experimental import pallas as pl
from jax.experimental.pallas import tpu as pltpu
from jax.experimental.pallas import tpu_sc as plsc

EPS = 1e-12
SUB = 128
SUBS_PER_CHUNK = 4
CHUNK = SUB * SUBS_PER_CHUNK


def _transpose_pack_body(x_ref, o_ref):
    D = x_ref.shape[0]
    y = x_ref[...].T
    y3 = y.reshape(y.shape[0] // 2, 2, D)
    o_ref[:, :D] = y3[:, 0, :]
    o_ref[:, D:] = y3[:, 1, :]


def _tc_transpose_pack(tT):
    D, V = tT.shape
    BN = 8192
    return pl.pallas_call(
        _transpose_pack_body,
        grid=(pl.cdiv(V, BN),),
        in_specs=[pl.BlockSpec((D, BN), lambda j: (0, j))],
        out_specs=pl.BlockSpec((BN // 2, 2 * D), lambda j: (j, 0)),
        out_shape=jax.ShapeDtypeStruct((V // 2, 2 * D), jnp.float32),
    )(tT)


def _sc_gather(table2, idx2d, n_rows):
    n_idx_rows, _ = idx2d.shape
    D = table2.shape[1]
    info = plsc.get_sparse_core_info()
    NC, NS = info.num_cores, info.num_subcores
    NW = NC * NS
    rows_pw = n_idx_rows // NW
    cpw = rows_pw // SUBS_PER_CHUNK
    assert cpw % 2 == 0
    mesh = plsc.VectorSubcoreMesh(core_axis_name="c", subcore_axis_name="s")

    @functools.partial(
        pl.kernel,
        mesh=mesh,
        out_type=jax.ShapeDtypeStruct((n_rows, D), jnp.float32),
        scratch_types=[
            pltpu.VMEM((SUBS_PER_CHUNK, SUB), jnp.int32),
            pltpu.VMEM((SUBS_PER_CHUNK, SUB), jnp.int32),
            pltpu.VMEM((CHUNK, D), jnp.float32),
            pltpu.VMEM((CHUNK, D), jnp.float32),
            pltpu.SemaphoreType.DMA,
            pltpu.SemaphoreType.DMA,
        ],
        compiler_params=pltpu.CompilerParams(use_tc_tiling_on_sc=False),
    )
    def k(tab_hbm, idx_hbm, out_hbm, idx0, idx1, rows0, rows1, sem0, sem1):
        wid = lax.axis_index("s") * NC + lax.axis_index("c")
        base = wid * rows_pw

        def fire(c, idx_v, rows_v, sem):
            r0 = base + c * SUBS_PER_CHUNK
            pltpu.sync_copy(idx_hbm.at[pl.ds(r0, SUBS_PER_CHUNK)], idx_v)
            for j in range(SUBS_PER_CHUNK):
                pltpu.async_copy(
                    tab_hbm.at[idx_v.at[j]],
                    rows_v.at[pl.ds(j * SUB, SUB)],
                    sem,
                )

        def drain_write(c, idx_v, rows_v, sem):
            for j in range(SUBS_PER_CHUNK):
                pltpu.make_async_copy(
                    tab_hbm.at[idx_v.at[j]],
                    rows_v.at[pl.ds(j * SUB, SUB)],
                    sem,
                ).wait()
            flat = (base + c * SUBS_PER_CHUNK) * SUB
            pltpu.sync_copy(rows_v, out_hbm.at[pl.ds(flat, CHUNK)])

        def body(i, carry):
            c_even = 2 * i
            fire(c_even, idx0, rows0, sem0)

            @pl.when(i > 0)
            def _():
                drain_write(c_even - 1, idx1, rows1, sem1)

            fire(c_even + 1, idx1, rows1, sem1)
            drain_write(c_even, idx0, rows0, sem0)
            return carry

        lax.fori_loop(0, cpw // 2, body, 0)
        drain_write(cpw - 1, idx1, rows1, sem1)

    return k(table2, idx2d)


def _ln_elu_body(x_ref, w_ref, b_ref, sel_ref, bc_ref, o_ref):
    x = x_ref[...]
    sel = sel_ref[...]
    bc = bc_ref[...]
    D = o_ref.shape[-1]
    inv = 1.0 / D
    sums = jax.lax.dot(x, sel, preferred_element_type=jnp.float32)
    u = jax.lax.dot(sums * inv, bc, preferred_element_type=jnp.float32)
    xc = x - u
    sq = jax.lax.dot(xc * xc, sel, preferred_element_type=jnp.float32)
    v = jax.lax.dot(sq * inv, bc, preferred_element_type=jnp.float32)
    y = xc * lax.rsqrt(v + EPS)
    y = y * w_ref[...] + b_ref[...]
    y = jnp.where(y > 0, y, jnp.exp(jnp.minimum(y, 0.0)) - 1.0)
    o_ref[::2, :] = y[:, :D]
    o_ref[1::2, :] = y[:, D:]


def _tc_ln_elu(x2, w2, b2, sel, bc):
    N2, L = x2.shape
    R = 4096
    return pl.pallas_call(
        _ln_elu_body,
        grid=(N2 // R,),
        in_specs=[
            pl.BlockSpec((R, L), lambda i: (i, 0)),
            pl.BlockSpec((1, L), lambda i: (0, 0)),
            pl.BlockSpec((1, L), lambda i: (0, 0)),
            pl.BlockSpec((L, 2), lambda i: (0, 0)),
            pl.BlockSpec((2, L), lambda i: (0, 0)),
        ],
        out_specs=pl.BlockSpec((2 * R, L // 2), lambda i: (i, 0)),
        out_shape=jax.ShapeDtypeStruct((2 * N2, L // 2), jnp.float32),
    )(x2, w2, b2, sel, bc)


def kernel(sequence, table, ln_weight, ln_bias):
    B, S = sequence.shape
    V, D = table.shape
    n_rows = B * S

    tpack = _tc_transpose_pack(table.T)
    table2 = tpack.reshape(V, D)

    idx2d = sequence.astype(jnp.int32).reshape(-1, SUB)
    g = _sc_gather(table2, idx2d, n_rows)
    g2 = g.reshape(n_rows // 2, 2 * D)

    half = jnp.arange(2 * D, dtype=jnp.int32) >= D
    sel = jnp.stack([1.0 - half.astype(jnp.float32),
                     half.astype(jnp.float32)], axis=1)
    bc = sel.T
    w2 = jnp.concatenate([ln_weight, ln_weight]).reshape(1, 2 * D)
    b2 = jnp.concatenate([ln_bias, ln_bias]).reshape(1, 2 * D)
    out = _tc_ln_elu(g2, w2, b2, sel, bc)
    return out.reshape(B, S, D)


# speedup vs baseline: 6.6306x; 1.0453x over previous
import functools

import jax
import jax.numpy as jnp
from jax import lax
from jax.experimental import pallas as pl
from jax.experimental.pallas import tpu as pltpu
from jax.experimental.pallas import tpu_sc as plsc

EPS = 1e-12
SUB = 128
SUBS_PER_CHUNK = 4
CHUNK = SUB * SUBS_PER_CHUNK


def _transpose_pack_body(x_ref, o_ref):
    D = x_ref.shape[0]
    y = x_ref[...].T
    y3 = y.reshape(y.shape[0] // 2, 2, D)
    o_ref[:, :D] = y3[:, 0, :]
    o_ref[:, D:] = y3[:, 1, :]


def _tc_transpose_pack(tT):
    D, V = tT.shape
    BN = 16384
    return pl.pallas_call(
        _transpose_pack_body,
        grid=(pl.cdiv(V, BN),),
        in_specs=[pl.BlockSpec((D, BN), lambda j: (0, j))],
        out_specs=pl.BlockSpec((BN // 2, 2 * D), lambda j: (j, 0)),
        out_shape=jax.ShapeDtypeStruct((V // 2, 2 * D), jnp.float32),
    )(tT)


def _sc_gather(table2, idx2d, n_rows):
    n_idx_rows, _ = idx2d.shape
    D = table2.shape[1]
    info = plsc.get_sparse_core_info()
    NC, NS = info.num_cores, info.num_subcores
    NW = NC * NS
    rows_pw = n_idx_rows // NW
    cpw = rows_pw // SUBS_PER_CHUNK
    assert cpw % 2 == 0
    mesh = plsc.VectorSubcoreMesh(core_axis_name="c", subcore_axis_name="s")

    @functools.partial(
        pl.kernel,
        mesh=mesh,
        out_type=jax.ShapeDtypeStruct((n_rows, D), jnp.float32),
        scratch_types=[
            pltpu.VMEM((SUBS_PER_CHUNK, SUB), jnp.int32),
            pltpu.VMEM((SUBS_PER_CHUNK, SUB), jnp.int32),
            pltpu.VMEM((CHUNK, D), jnp.float32),
            pltpu.VMEM((CHUNK, D), jnp.float32),
            pltpu.SemaphoreType.DMA,
            pltpu.SemaphoreType.DMA,
        ],
        compiler_params=pltpu.CompilerParams(use_tc_tiling_on_sc=False),
    )
    def k(tab_hbm, idx_hbm, out_hbm, idx0, idx1, rows0, rows1, sem0, sem1):
        wid = lax.axis_index("s") * NC + lax.axis_index("c")
        base = wid * rows_pw

        def fire(c, idx_v, rows_v, sem):
            r0 = base + c * SUBS_PER_CHUNK
            pltpu.sync_copy(idx_hbm.at[pl.ds(r0, SUBS_PER_CHUNK)], idx_v)
            for j in range(SUBS_PER_CHUNK):
                pltpu.async_copy(
                    tab_hbm.at[idx_v.at[j]],
                    rows_v.at[pl.ds(j * SUB, SUB)],
                    sem,
                )

        def drain_write(c, idx_v, rows_v, sem):
            for j in range(SUBS_PER_CHUNK):
                pltpu.make_async_copy(
                    tab_hbm.at[idx_v.at[j]],
                    rows_v.at[pl.ds(j * SUB, SUB)],
                    sem,
                ).wait()
            flat = (base + c * SUBS_PER_CHUNK) * SUB
            pltpu.sync_copy(rows_v, out_hbm.at[pl.ds(flat, CHUNK)])

        def body(i, carry):
            c_even = 2 * i
            fire(c_even, idx0, rows0, sem0)

            @pl.when(i > 0)
            def _():
                drain_write(c_even - 1, idx1, rows1, sem1)

            fire(c_even + 1, idx1, rows1, sem1)
            drain_write(c_even, idx0, rows0, sem0)
            return carry

        lax.fori_loop(0, cpw // 2, body, 0)
        drain_write(cpw - 1, idx1, rows1, sem1)

    return k(table2, idx2d)


def _ln_elu_body(x_ref, w_ref, b_ref, sel_ref, bc_ref, o_ref):
    x = x_ref[...]
    sel = sel_ref[...]
    bc = bc_ref[...]
    D = o_ref.shape[-1]
    inv = 1.0 / D
    sums = jax.lax.dot(x, sel, preferred_element_type=jnp.float32)
    u = jax.lax.dot(sums * inv, bc, preferred_element_type=jnp.float32)
    xc = x - u
    sq = jax.lax.dot(xc * xc, sel, preferred_element_type=jnp.float32)
    v = jax.lax.dot(sq * inv, bc, preferred_element_type=jnp.float32)
    y = xc * lax.rsqrt(v + EPS)
    y = y * w_ref[...] + b_ref[...]
    y = jnp.where(y > 0, y, jnp.exp(jnp.minimum(y, 0.0)) - 1.0)
    o_ref[::2, :] = y[:, :D]
    o_ref[1::2, :] = y[:, D:]


def _tc_ln_elu(x2, w2, b2, sel, bc):
    N2, L = x2.shape
    R = 8192
    return pl.pallas_call(
        _ln_elu_body,
        grid=(N2 // R,),
        in_specs=[
            pl.BlockSpec((R, L), lambda i: (i, 0)),
            pl.BlockSpec((1, L), lambda i: (0, 0)),
            pl.BlockSpec((1, L), lambda i: (0, 0)),
            pl.BlockSpec((L, 2), lambda i: (0, 0)),
            pl.BlockSpec((2, L), lambda i: (0, 0)),
        ],
        out_specs=pl.BlockSpec((2 * R, L // 2), lambda i: (i, 0)),
        out_shape=jax.ShapeDtypeStruct((2 * N2, L // 2), jnp.float32),
    )(x2, w2, b2, sel, bc)


def kernel(sequence, table, ln_weight, ln_bias):
    B, S = sequence.shape
    V, D = table.shape
    n_rows = B * S

    tpack = _tc_transpose_pack(table.T)
    table2 = tpack.reshape(V, D)

    idx2d = sequence.astype(jnp.int32).reshape(-1, SUB)
    g = _sc_gather(table2, idx2d, n_rows)
    g2 = g.reshape(n_rows // 2, 2 * D)

    half = jnp.arange(2 * D, dtype=jnp.int32) >= D
    sel = jnp.stack([1.0 - half.astype(jnp.float32),
                     half.astype(jnp.float32)], axis=1)
    bc = sel.T
    w2 = jnp.concatenate([ln_weight, ln_weight]).reshape(1, 2 * D)
    b2 = jnp.concatenate([ln_bias, ln_bias]).reshape(1, 2 * D)
    out = _tc_ln_elu(g2, w2, b2, sel, bc)
    return out.reshape(B, S, D)
